# baseline (speedup 1.0000x reference)
import functools

import jax
import jax.numpy as jnp
from jax import lax
from jax.experimental import pallas as pl
from jax.experimental.pallas import tpu as pltpu
from jax.experimental.pallas import tpu_sc as plsc

N = 10000
E = 320000
F = 16
G = 200
NC = 2
NS = 16
NW = NC * NS
NP = 10240
RPT = NP // NS
EPT = E // NW
GPT = EPT // G
ER = E // G
SPT = NP // NW
D = 12
NBUF = 2 * D
NPW = NP // 8

_MESH = plsc.VectorSubcoreMesh(core_axis_name="c", subcore_axis_name="s")


def _zero_acc_slice(zbuf, acc, s):
    def zb(i, carry):
        zbuf[i, :] = jnp.zeros((F,), jnp.float32)
        return carry
    lax.fori_loop(0, RPT, zb, 0)
    pltpu.sync_copy(zbuf, acc.at[pl.ds(s * RPT, RPT)])


def _gen_self_idx(selfd, wid):
    base = wid * SPT
    def sb(i, carry):
        selfd[pl.ds(i * F, F)] = base + i * F + lax.iota(jnp.int32, F)
        return carry
    lax.fori_loop(0, SPT // F, sb, 0)


def _copy_out(acc, out_hbm, c, s):
    pltpu.sync_copy(acc.at[pl.ds(s * RPT, RPT)],
                    out_hbm.at[c, pl.ds(s * RPT, RPT)])


@functools.partial(
    pl.kernel,
    out_type=jax.ShapeDtypeStruct((NC, NP, F), jnp.float32),
    mesh=_MESH,
    compiler_params=pltpu.CompilerParams(use_tc_tiling_on_sc=False),
    scratch_types=[
        pltpu.VMEM((EPT,), jnp.int32),
        pltpu.VMEM((SPT, F), jnp.float32),
        pltpu.VMEM((SPT,), jnp.int32),
        pltpu.VMEM((RPT, F), jnp.float32),
        pltpu.VMEM_SHARED((NP, F), jnp.float32),
        pltpu.SemaphoreType.DMA,
    ],
)
def _deg_pass(dst_hbm, out_hbm, dstv, ones_v, selfd, zbuf, acc, ssem):
    c = lax.axis_index("c")
    s = lax.axis_index("s")
    wid = c * NS + s
    _zero_acc_slice(zbuf, acc, s)

    def ob(i, carry):
        ones_v[i, :] = jnp.ones((F,), jnp.float32)
        return carry
    lax.fori_loop(0, SPT, ob, 0)
    _gen_self_idx(selfd, wid)
    pltpu.sync_copy(dst_hbm.at[pl.ds(wid * EPT, EPT)], dstv)
    plsc.subcore_barrier()

    ones_g = ones_v.at[pl.ds(0, G)]

    def grp(j, carry):
        @pl.when(j >= D)
        def _():
            pltpu.make_async_copy(
                ones_g, acc.at[dstv.at[pl.ds((j - D) * G, G)]], ssem).wait()
        pltpu.async_copy(ones_g, acc.at[dstv.at[pl.ds(j * G, G)]], ssem,
                         add=True)
        return carry
    lax.fori_loop(0, GPT, grp, 0)
    pltpu.async_copy(ones_v, acc.at[selfd], ssem, add=True)
    for i in range(D):
        pltpu.make_async_copy(
            ones_g, acc.at[dstv.at[pl.ds((GPT - D + i) * G, G)]], ssem).wait()
    pltpu.make_async_copy(ones_v, acc.at[selfd], ssem).wait()
    plsc.subcore_barrier()
    _copy_out(acc, out_hbm, c, s)


@functools.partial(
    pl.kernel,
    out_type=jax.ShapeDtypeStruct((NC, NP, F), jnp.float32),
    mesh=_MESH,
    compiler_params=pltpu.CompilerParams(use_tc_tiling_on_sc=False),
    scratch_types=[
        pltpu.VMEM((EPT,), jnp.int32),
        pltpu.VMEM((EPT,), jnp.int32),
        pltpu.VMEM((NBUF, G, F), jnp.float32),
        pltpu.VMEM((SPT, F), jnp.float32),
        pltpu.VMEM((SPT,), jnp.int32),
        pltpu.VMEM((RPT, F), jnp.float32),
        pltpu.VMEM_SHARED((NP, F), jnp.float32),
        pltpu.SemaphoreType.DMA,
        pltpu.SemaphoreType.DMA,
    ],
)
def _prop_pass(z_hbm, src_hbm, dst_hbm, out_hbm, srcv, dstv, rows, selfr,
               selfd, zbuf, acc, gsem, ssem):
    c = lax.axis_index("c")
    s = lax.axis_index("s")
    wid = c * NS + s
    _zero_acc_slice(zbuf, acc, s)
    pltpu.sync_copy(src_hbm.at[pl.ds(wid * EPT, EPT)], srcv)
    pltpu.sync_copy(dst_hbm.at[pl.ds(wid * EPT, EPT)], dstv)
    pltpu.sync_copy(z_hbm.at[pl.ds(wid * SPT, SPT)], selfr)
    _gen_self_idx(selfd, wid)
    plsc.subcore_barrier()

    for b in range(D):
        pltpu.async_copy(z_hbm.at[srcv.at[pl.ds(b * G, G)]], rows.at[b], gsem)

    def grp(j, carry):
        jb = lax.rem(j, NBUF)

        @pl.when(j >= D)
        def _():
            jd = j - D
            pltpu.make_async_copy(rows.at[lax.rem(jd, NBUF)],
                                  acc.at[dstv.at[pl.ds(jd * G, G)]],
                                  ssem).wait()
        pltpu.make_async_copy(z_hbm.at[srcv.at[pl.ds(j * G, G)]], rows.at[jb],
                              gsem).wait()
        pltpu.async_copy(rows.at[jb], acc.at[dstv.at[pl.ds(j * G, G)]], ssem,
                         add=True)

        @pl.when(j + D < GPT)
        def _():
            jn = j + D
            pltpu.async_copy(z_hbm.at[srcv.at[pl.ds(jn * G, G)]],
                             rows.at[lax.rem(jn, NBUF)], gsem)
        return carry
    lax.fori_loop(0, GPT, grp, 0)
    pltpu.async_copy(selfr, acc.at[selfd], ssem, add=True)
    for i in range(D):
        jd = GPT - D + i
        pltpu.make_async_copy(rows.at[jd % NBUF],
                              acc.at[dstv.at[pl.ds(jd * G, G)]], ssem).wait()
    pltpu.make_async_copy(selfr, acc.at[selfd], ssem).wait()
    plsc.subcore_barrier()
    _copy_out(acc, out_hbm, c, s)


def _dA_body(xw_ref, w1_ref, u1_ref):
    u1_ref[...] = jnp.dot(xw_ref[...], w1_ref[...],
                          preferred_element_type=jnp.float32)


def _dB_body(degw_ref, u1_ref, z1_ref, dinv_ref):
    dinv = lax.rsqrt(degw_ref[0] + degw_ref[1])
    dinv_ref[...] = dinv
    z1_ref[...] = dinv * u1_ref[...]


def _d1_body(p_ref, dinv_ref, w_ref, b_ref, z2_ref):
    dinv = dinv_ref[...]
    h = jnp.maximum(dinv * (p_ref[0] + p_ref[1]) + b_ref[...], 0.0)
    z2_ref[...] = dinv * jnp.dot(h, w_ref[...],
                                 preferred_element_type=jnp.float32)


def _d2_body(p_ref, dinv_ref, b_ref, z3_ref):
    dinv = dinv_ref[...]
    h = jnp.maximum(dinv * (p_ref[0] + p_ref[1]) + b_ref[...], 0.0)
    z3_ref[...] = dinv * h


def _d3_body(p_ref, dinv_ref, w_ref, b_ref, out_ref):
    t = dinv_ref[0:N // 8] * (p_ref[0, 0:N // 8] + p_ref[1, 0:N // 8])
    res = jnp.dot(t, w_ref[...],
                  preferred_element_type=jnp.float32) + b_ref[...]
    out_ref[...] = res.reshape(N, res.shape[1] // 8)


def _f32(shape):
    return jax.ShapeDtypeStruct(shape, jnp.float32)


def kernel(x, edge_index, W1, b1, W2, b2, W3, b3):
    ei = edge_index.astype(jnp.int32)
    src = ei[0]
    dst = ei[1]

    xw = jnp.pad(x, ((0, NP - N), (0, 0))).reshape(NPW, 8 * x.shape[1])
    eye8 = jnp.eye(8, dtype=jnp.float32)
    W1big = jnp.kron(eye8, W1)
    W2bd = jnp.kron(eye8, W2)
    W3big = jnp.kron(eye8, W3)
    b1w = jnp.tile(b1, 8)[None]
    b2w = jnp.tile(b2, 8)[None]
    b3w = jnp.tile(b3, 8)[None]

    degp = _deg_pass(dst)
    u1w = pl.pallas_call(_dA_body, out_shape=_f32((NPW, 128)))(xw, W1big)
    z1w, dinvw = pl.pallas_call(
        _dB_body, out_shape=[_f32((NPW, 128)), _f32((NPW, 128))],
    )(degp.reshape(NC, NPW, 128), u1w)

    p1 = _prop_pass(z1w.reshape(NP, F), src, dst)
    z2w = pl.pallas_call(_d1_body, out_shape=_f32((NPW, 128)))(
        p1.reshape(NC, NPW, 128), dinvw, W2bd, b1w)

    p2 = _prop_pass(z2w.reshape(NP, F), src, dst)
    z3w = pl.pallas_call(_d2_body, out_shape=_f32((NPW, 128)))(
        p2.reshape(NC, NPW, 128), dinvw, b2w)

    p3 = _prop_pass(z3w.reshape(NP, F), src, dst)
    out = pl.pallas_call(_d3_body, out_shape=_f32((N, W3.shape[1])))(
        p3.reshape(NC, NPW, 128), dinvw, W3big, b3w)
    return out


# speedup vs baseline: 1.0960x; 1.0960x over previous
import functools

import jax
import jax.numpy as jnp
from jax import lax
from jax.experimental import pallas as pl
from jax.experimental.pallas import tpu as pltpu
from jax.experimental.pallas import tpu_sc as plsc

N = 10000
E = 320000
F = 16
G = 200
NC = 2
NS = 16
NW = NC * NS
NP = 10240
RPT = NP // NS
EPT = E // NW
GPT = EPT // G
ER = E // G
SPT = NP // NW
D = 8
NBUF = 2 * D
NPW = NP // 8

_MESH = plsc.VectorSubcoreMesh(core_axis_name="c", subcore_axis_name="s")


def _zero_acc_slice(zbuf, acc, s):
    def zb(i, carry):
        zbuf[i, :] = jnp.zeros((F,), jnp.float32)
        return carry
    lax.fori_loop(0, RPT, zb, 0)
    pltpu.sync_copy(zbuf, acc.at[pl.ds(s * RPT, RPT)])


def _gen_self_idx(selfd, wid):
    base = wid * SPT
    def sb(i, carry):
        selfd[pl.ds(i * F, F)] = base + i * F + lax.iota(jnp.int32, F)
        return carry
    lax.fori_loop(0, SPT // F, sb, 0)


def _copy_out(acc, out_hbm, c, s):
    pltpu.sync_copy(acc.at[pl.ds(s * RPT, RPT)],
                    out_hbm.at[c, pl.ds(s * RPT, RPT)])


@functools.partial(
    pl.kernel,
    out_type=jax.ShapeDtypeStruct((NC, NP, F), jnp.float32),
    mesh=_MESH,
    compiler_params=pltpu.CompilerParams(use_tc_tiling_on_sc=False),
    scratch_types=[
        pltpu.VMEM((EPT,), jnp.int32),
        pltpu.VMEM((SPT, F), jnp.float32),
        pltpu.VMEM((SPT,), jnp.int32),
        pltpu.VMEM((RPT, F), jnp.float32),
        pltpu.VMEM_SHARED((NP, F), jnp.float32),
        pltpu.SemaphoreType.DMA,
    ],
)
def _deg_pass(ei_hbm, out_hbm, dstv, ones_v, selfd, zbuf, acc, ssem):
    c = lax.axis_index("c")
    s = lax.axis_index("s")
    wid = c * NS + s
    _zero_acc_slice(zbuf, acc, s)

    def ob(i, carry):
        ones_v[i, :] = jnp.ones((F,), jnp.float32)
        return carry
    lax.fori_loop(0, SPT, ob, 0)
    _gen_self_idx(selfd, wid)
    pltpu.sync_copy(ei_hbm.at[1, pl.ds(wid * EPT, EPT)], dstv)
    plsc.subcore_barrier()

    ones_g = ones_v.at[pl.ds(0, G)]

    def grp(j, carry):
        @pl.when(j >= D)
        def _():
            pltpu.make_async_copy(
                ones_g, acc.at[dstv.at[pl.ds((j - D) * G, G)]], ssem).wait()
        pltpu.async_copy(ones_g, acc.at[dstv.at[pl.ds(j * G, G)]], ssem,
                         add=True)
        return carry
    lax.fori_loop(0, GPT, grp, 0)
    pltpu.async_copy(ones_v, acc.at[selfd], ssem, add=True)
    for i in range(D):
        pltpu.make_async_copy(
            ones_g, acc.at[dstv.at[pl.ds((GPT - D + i) * G, G)]], ssem).wait()
    pltpu.make_async_copy(ones_v, acc.at[selfd], ssem).wait()
    plsc.subcore_barrier()
    _copy_out(acc, out_hbm, c, s)


@functools.partial(
    pl.kernel,
    out_type=jax.ShapeDtypeStruct((NC, NP, F), jnp.float32),
    mesh=_MESH,
    compiler_params=pltpu.CompilerParams(use_tc_tiling_on_sc=False),
    scratch_types=[
        pltpu.VMEM((EPT,), jnp.int32),
        pltpu.VMEM((EPT,), jnp.int32),
        pltpu.VMEM((NBUF, G, F), jnp.float32),
        pltpu.VMEM((SPT, F), jnp.float32),
        pltpu.VMEM((SPT,), jnp.int32),
        pltpu.VMEM((RPT, F), jnp.float32),
        pltpu.VMEM_SHARED((NP, F), jnp.float32),
        pltpu.SemaphoreType.DMA,
        pltpu.SemaphoreType.DMA,
    ],
)
def _prop_pass(z_hbm, ei_hbm, out_hbm, srcv, dstv, rows, selfr,
               selfd, zbuf, acc, gsem, ssem):
    c = lax.axis_index("c")
    s = lax.axis_index("s")
    wid = c * NS + s
    _zero_acc_slice(zbuf, acc, s)
    pltpu.sync_copy(ei_hbm.at[0, pl.ds(wid * EPT, EPT)], srcv)
    pltpu.sync_copy(ei_hbm.at[1, pl.ds(wid * EPT, EPT)], dstv)
    pltpu.sync_copy(z_hbm.at[pl.ds(wid * SPT, SPT)], selfr)
    _gen_self_idx(selfd, wid)
    plsc.subcore_barrier()

    for b in range(D):
        pltpu.async_copy(z_hbm.at[srcv.at[pl.ds(b * G, G)]], rows.at[b], gsem)

    def grp(j, carry):
        jb = lax.rem(j, NBUF)

        @pl.when(j >= D)
        def _():
            jd = j - D
            pltpu.make_async_copy(rows.at[lax.rem(jd, NBUF)],
                                  acc.at[dstv.at[pl.ds(jd * G, G)]],
                                  ssem).wait()
        pltpu.make_async_copy(z_hbm.at[srcv.at[pl.ds(j * G, G)]], rows.at[jb],
                              gsem).wait()
        pltpu.async_copy(rows.at[jb], acc.at[dstv.at[pl.ds(j * G, G)]], ssem,
                         add=True)

        @pl.when(j + D < GPT)
        def _():
            jn = j + D
            pltpu.async_copy(z_hbm.at[srcv.at[pl.ds(jn * G, G)]],
                             rows.at[lax.rem(jn, NBUF)], gsem)
        return carry
    lax.fori_loop(0, GPT, grp, 0)
    pltpu.async_copy(selfr, acc.at[selfd], ssem, add=True)
    for i in range(D):
        jd = GPT - D + i
        pltpu.make_async_copy(rows.at[jd % NBUF],
                              acc.at[dstv.at[pl.ds(jd * G, G)]], ssem).wait()
    pltpu.make_async_copy(selfr, acc.at[selfd], ssem).wait()
    plsc.subcore_barrier()
    _copy_out(acc, out_hbm, c, s)


def _dA_body(xw_ref, w1_ref, u1_ref):
    u1_ref[...] = jnp.dot(xw_ref[...], w1_ref[...],
                          preferred_element_type=jnp.float32)


def _dB_body(degw_ref, u1_ref, z1_ref, dinv_ref):
    dinv = lax.rsqrt(degw_ref[0] + degw_ref[1])
    dinv_ref[...] = dinv
    z1_ref[...] = dinv * u1_ref[...]


def _d1_body(p_ref, dinv_ref, w_ref, b_ref, z2_ref):
    dinv = dinv_ref[...]
    h = jnp.maximum(dinv * (p_ref[0] + p_ref[1]) + b_ref[...], 0.0)
    z2_ref[...] = dinv * jnp.dot(h, w_ref[...],
                                 preferred_element_type=jnp.float32)


def _d2_body(p_ref, dinv_ref, b_ref, z3_ref):
    dinv = dinv_ref[...]
    h = jnp.maximum(dinv * (p_ref[0] + p_ref[1]) + b_ref[...], 0.0)
    z3_ref[...] = dinv * h


def _d3_body(p_ref, dinv_ref, w_ref, b_ref, out_ref):
    t = dinv_ref[0:N // 8] * (p_ref[0, 0:N // 8] + p_ref[1, 0:N // 8])
    res = jnp.dot(t, w_ref[...],
                  preferred_element_type=jnp.float32) + b_ref[...]
    out_ref[...] = res.reshape(N, res.shape[1] // 8)


def _f32(shape):
    return jax.ShapeDtypeStruct(shape, jnp.float32)


def kernel(x, edge_index, W1, b1, W2, b2, W3, b3):
    ei = edge_index.astype(jnp.int32)

    xw = jnp.pad(x, ((0, NP - N), (0, 0))).reshape(NPW, 8 * x.shape[1])
    eye8 = jnp.eye(8, dtype=jnp.float32)
    W1big = jnp.kron(eye8, W1)
    W2bd = jnp.kron(eye8, W2)
    W3big = jnp.kron(eye8, W3)
    b1w = jnp.tile(b1, 8)[None]
    b2w = jnp.tile(b2, 8)[None]
    b3w = jnp.tile(b3, 8)[None]

    degp = _deg_pass(ei)
    u1w = pl.pallas_call(_dA_body, out_shape=_f32((NPW, 128)))(xw, W1big)
    z1w, dinvw = pl.pallas_call(
        _dB_body, out_shape=[_f32((NPW, 128)), _f32((NPW, 128))],
    )(degp.reshape(NC, NPW, 128), u1w)

    p1 = _prop_pass(z1w.reshape(NP, F), ei)
    z2w = pl.pallas_call(_d1_body, out_shape=_f32((NPW, 128)))(
        p1.reshape(NC, NPW, 128), dinvw, W2bd, b1w)

    p2 = _prop_pass(z2w.reshape(NP, F), ei)
    z3w = pl.pallas_call(_d2_body, out_shape=_f32((NPW, 128)))(
        p2.reshape(NC, NPW, 128), dinvw, b2w)

    p3 = _prop_pass(z3w.reshape(NP, F), ei)
    out = pl.pallas_call(_d3_body, out_shape=_f32((N, W3.shape[1])))(
        p3.reshape(NC, NPW, 128), dinvw, W3big, b3w)
    return out
